# Initial kernel scaffold; baseline (speedup 1.0000x reference)
import jax
import jax.numpy as jnp
from jax.experimental import pallas as pl


def kernel(x, w1p, b1p, w2p, b2p, w3p, b3p):
    raise NotImplementedError("write your pallas kernel here")


# speedup vs baseline: 1.4338x; 1.4338x over previous
import functools

import jax
import jax.numpy as jnp
from jax.experimental import pallas as pl
from jax.experimental.pallas import tpu as pltpu

_NEG_SLOPE = 0.01
_F = 32
_C = 2
_G = 8
_S = 4096


def _leaky(h):
    return jnp.where(h > 0, h, _NEG_SLOPE * h)


def _packed_kernel(x_ref, w1_ref, b1_ref, w2_ref, b2_ref, w3_ref, b3_ref,
                   o_ref):
    xg = x_ref[...]
    s = xg.shape[0]
    xl = jnp.concatenate(
        [jnp.broadcast_to(xg[:, g:g + 1], (s, _F)) for g in range(_G)],
        axis=1)
    h1 = _leaky(xl * w1_ref[...] + b1_ref[...])
    h2 = jnp.dot(h1, w2_ref[...],
                 preferred_element_type=jnp.float32) + b2_ref[...]
    h2 = _leaky(h2)
    h3 = jnp.dot(h2, w3_ref[...],
                 preferred_element_type=jnp.float32)
    gc = _G * _C
    o_ref[...] = jnp.tanh(h3[:, :gc] + b3_ref[0:1, :gc])


@jax.jit
def kernel(x, w1p, b1p, w2p, b2p, w3p, b3p):
    B = x.shape[0]
    f32 = jnp.float32
    w1 = w1p[0, :_F].astype(f32)
    b1 = b1p[0, :_F].astype(f32)
    w2 = w2p[:_F, :_F].astype(f32)
    b2 = b2p[0, :_F].astype(f32)
    w3 = w3p[:_F, :_C].astype(f32)
    b3 = b3p[0, :_C].astype(f32)

    K = _G * _F
    gc = _G * _C
    w1t = jnp.tile(w1, _G)[None, :]
    b1t = jnp.tile(b1, _G)[None, :]
    w2bd = jnp.kron(jnp.eye(_G, dtype=f32), w2)
    b2t = jnp.tile(b2, _G)[None, :]
    w3bd = jnp.kron(jnp.eye(_G, dtype=f32), w3)
    w3f = jnp.zeros((K, 128), f32).at[:, :gc].set(w3bd)
    b3t = jnp.zeros((1, 128), f32).at[0, :gc].set(jnp.tile(b3, _G))

    nb = pl.cdiv(B, _G * _S)
    nb += nb % 2
    rows = nb * _S
    xf = x.reshape(-1).astype(f32)
    if rows * _G != B:
        xf = jnp.pad(xf, (0, rows * _G - B))
    xg = xf.reshape(rows, _G)

    def const(shape):
        return pl.BlockSpec(shape, lambda i: (0, 0))

    cost = pl.CostEstimate(
        flops=2 * rows * K * (K + 128) + 2 * rows * K,
        transcendentals=rows * gc,
        bytes_accessed=4 * (rows * _G + K * (K + 128) + rows * gc),
    )

    out = pl.pallas_call(
        _packed_kernel,
        out_shape=jax.ShapeDtypeStruct((rows, gc), f32),
        grid_spec=pltpu.PrefetchScalarGridSpec(
            num_scalar_prefetch=0,
            grid=(nb,),
            in_specs=[
                pl.BlockSpec((_S, _G), lambda i: (i, 0)),
                const((1, K)), const((1, K)),
                const((K, K)), const((1, K)),
                const((K, 128)), const((1, 128)),
            ],
            out_specs=pl.BlockSpec((_S, gc), lambda i: (i, 0)),
        ),
        compiler_params=pltpu.CompilerParams(
            dimension_semantics=("parallel",),
            vmem_limit_bytes=48 * 1024 * 1024,
        ),
        cost_estimate=cost,
    )(xg, w1t, b1t, w2bd, b2t, w3f, b3t)

    return out.reshape(rows * _G, _C)[:B]
